# baseline (speedup 1.0000x reference)
import functools

import jax
import jax.numpy as jnp
from jax import lax
from jax.experimental import pallas as pl
from jax.experimental.pallas import tpu as pltpu
from jax.experimental.pallas import tpu_sc as plsc

NC, NS = 2, 16
NW = NC * NS
L = 16
ROWS, COLS = 16384, 200
FULL = (COLS // L) * L
ROWS_W = ROWS // NW
CHUNK_R = 128
NCHUNK = ROWS_W // CHUNK_R


@functools.cache
def _build_sc_partial_sums():
    mesh = plsc.VectorSubcoreMesh(
        core_axis_name="c", subcore_axis_name="s", num_cores=NC, num_subcores=NS
    )
    return functools.partial(
        pl.kernel,
        out_type=jax.ShapeDtypeStruct((NW, 2 * L), jnp.float32),
        mesh=mesh,
        compiler_params=pltpu.CompilerParams(use_tc_tiling_on_sc=True),
        scratch_types=[
            pltpu.VMEM((CHUNK_R, COLS), jnp.float32),
            pltpu.VMEM((CHUNK_R, COLS), jnp.float32),
            pltpu.VMEM((L,), jnp.float32),
            pltpu.VMEM((2 * L,), jnp.float32),
        ],
    )(_sc_partial_sums)


def _sc_partial_sums(p_hbm, t_hbm, v_hbm, out_hbm, p_v, t_v, v_v, part_v):
    wid = lax.axis_index("s") * NC + lax.axis_index("c")
    base = wid * ROWS_W
    pltpu.sync_copy(v_hbm, v_v)
    v = v_v[...]
    num = jnp.zeros((L,), jnp.float32)
    cnt = jnp.zeros((L,), jnp.float32)
    def acc(t, p, num, cnt, tail=False):
        if tail:
            t = jnp.where(lax.iota(jnp.int32, L) >= (L - (COLS - FULL)), t, v)
        m = t > v
        safe = jnp.where(m, t, jnp.inf)
        num = num + jnp.abs((t - p) / safe)
        cnt = cnt + jnp.where(m, 1.0, 0.0)
        return num, cnt

    for c in range(NCHUNK):
        r0 = base + c * CHUNK_R
        pltpu.sync_copy(p_hbm.at[pl.ds(r0, CHUNK_R)], p_v)
        pltpu.sync_copy(t_hbm.at[pl.ds(r0, CHUNK_R)], t_v)

        def rows_body(r, carry, p_v=p_v, t_v=t_v):
            num, cnt = carry
            for j in range(FULL // L):
                t = t_v[r, pl.ds(j * L, L)]
                p = p_v[r, pl.ds(j * L, L)]
                num, cnt = acc(t, p, num, cnt)
            t = t_v[r, pl.ds(COLS - L, L)]
            p = p_v[r, pl.ds(COLS - L, L)]
            num, cnt = acc(t, p, num, cnt, tail=True)
            return num, cnt

        num, cnt = lax.fori_loop(0, CHUNK_R, rows_body, (num, cnt))
    part_v[pl.ds(0, L)] = num
    part_v[pl.ds(L, L)] = cnt
    pltpu.sync_copy(part_v, out_hbm.at[wid])


def _combine_body(parts_ref, o_ref):
    x = parts_ref[...]
    num = jnp.sum(x[:, :L])
    cnt = jnp.sum(x[:, L:])
    o_ref[...] = jnp.broadcast_to(num / cnt, (1, 1))


def kernel(preds, targets, value):
    v_vec = jnp.full((L,), jnp.asarray(value, jnp.float32))
    parts = _build_sc_partial_sums()(preds, targets, v_vec)
    out = pl.pallas_call(
        _combine_body,
        out_shape=jax.ShapeDtypeStruct((1, 1), jnp.float32),
    )(parts)
    return out[0, 0]


# speedup vs baseline: 1.3888x; 1.3888x over previous
import functools

import jax
import jax.numpy as jnp
from jax import lax
from jax.experimental import pallas as pl
from jax.experimental.pallas import tpu as pltpu
from jax.experimental.pallas import tpu_sc as plsc

NC, NS = 2, 16
NW = NC * NS
L = 16
ROWS_T, COLS_T = 200, 16384
COLS_W = COLS_T // NW
CCHUNK = 128
NCHUNK = COLS_W // CCHUNK
RQUAD = 4
VPR = CCHUNK // L


@functools.cache
def _build_sc_partial_sums():
    mesh = plsc.VectorSubcoreMesh(
        core_axis_name="c", subcore_axis_name="s", num_cores=NC, num_subcores=NS
    )
    return functools.partial(
        pl.kernel,
        out_type=jax.ShapeDtypeStruct((NW, 2 * L), jnp.float32),
        mesh=mesh,
        scratch_types=[
            pltpu.VMEM((ROWS_T, CCHUNK), jnp.float32),
            pltpu.VMEM((ROWS_T, CCHUNK), jnp.float32),
            pltpu.VMEM((ROWS_T, CCHUNK), jnp.float32),
            pltpu.VMEM((ROWS_T, CCHUNK), jnp.float32),
            pltpu.VMEM((L,), jnp.float32),
            pltpu.VMEM((2 * L,), jnp.float32),
            pltpu.SemaphoreType.DMA,
            pltpu.SemaphoreType.DMA,
            pltpu.SemaphoreType.DMA,
            pltpu.SemaphoreType.DMA,
        ],
    )(_sc_partial_sums)


def _sc_partial_sums(
    p_hbm, t_hbm, v_hbm, out_hbm, p0, p1, t0, t1, v_v, part_v, s0, s1, s2, s3
):
    wid = lax.axis_index("s") * NC + lax.axis_index("c")
    col0 = wid * COLS_W
    pltpu.sync_copy(v_hbm, v_v)
    v = v_v[...]
    bufs = ((p0, t0, s0, s1), (p1, t1, s2, s3))

    def start(c):
        pb, tb, sp, st = bufs[c % 2]
        cols = pl.ds(col0 + c * CCHUNK, CCHUNK)
        cp = pltpu.make_async_copy(p_hbm.at[:, cols], pb, sp)
        ct = pltpu.make_async_copy(t_hbm.at[:, cols], tb, st)
        cp.start()
        ct.start()
        return cp, ct

    num = jnp.zeros((L,), jnp.float32)
    cnt = jnp.zeros((L,), jnp.float32)
    pending = start(0)
    for c in range(NCHUNK):
        if c + 1 < NCHUNK:
            nxt = start(c + 1)
        pending[0].wait()
        pending[1].wait()
        pb, tb, _, _ = bufs[c % 2]

        def quad(i, carry, pb=pb, tb=tb):
            num, cnt = carry
            for rr in range(RQUAD):
                r = i * RQUAD + rr
                for j in range(VPR):
                    t = tb[r, pl.ds(j * L, L)]
                    p = pb[r, pl.ds(j * L, L)]
                    m = t > v
                    safe = jnp.where(m, t, jnp.inf)
                    num = num + jnp.abs((t - p) / safe)
                    cnt = cnt + jnp.where(m, 1.0, 0.0)
            return num, cnt

        num, cnt = lax.fori_loop(0, ROWS_T // RQUAD, quad, (num, cnt))
        if c + 1 < NCHUNK:
            pending = nxt
    part_v[pl.ds(0, L)] = num
    part_v[pl.ds(L, L)] = cnt
    pltpu.sync_copy(part_v, out_hbm.at[wid])


def _combine_body(parts_ref, o_ref):
    x = parts_ref[...]
    num = jnp.sum(x[:, :L])
    cnt = jnp.sum(x[:, L:])
    o_ref[...] = jnp.broadcast_to(num / cnt, (1, 1))


def kernel(preds, targets, value):
    v_vec = jnp.full((L,), jnp.asarray(value, jnp.float32))
    parts = _build_sc_partial_sums()(preds.T, targets.T, v_vec)
    out = pl.pallas_call(
        _combine_body,
        out_shape=jax.ShapeDtypeStruct((1, 1), jnp.float32),
    )(parts)
    return out[0, 0]


# speedup vs baseline: 1.9710x; 1.4192x over previous
import functools

import jax
import jax.numpy as jnp
from jax import lax
from jax.experimental import pallas as pl
from jax.experimental.pallas import tpu as pltpu
from jax.experimental.pallas import tpu_sc as plsc

NC, NS = 2, 16
NW = NC * NS
L = 16
ROWS_T, COLS_T = 200, 16384
COLS_W = COLS_T // NW
CCHUNK = 128
NCHUNK = COLS_W // CCHUNK
RQUAD = 2
VPR = CCHUNK // L


@functools.cache
def _build_sc_partial_sums():
    mesh = plsc.VectorSubcoreMesh(
        core_axis_name="c", subcore_axis_name="s", num_cores=NC, num_subcores=NS
    )
    return functools.partial(
        pl.kernel,
        out_type=jax.ShapeDtypeStruct((NW, 2 * L), jnp.float32),
        mesh=mesh,
        scratch_types=[
            pltpu.VMEM((ROWS_T, CCHUNK), jnp.float32),
            pltpu.VMEM((ROWS_T, CCHUNK), jnp.float32),
            pltpu.VMEM((ROWS_T, CCHUNK), jnp.float32),
            pltpu.VMEM((ROWS_T, CCHUNK), jnp.float32),
            pltpu.VMEM((L,), jnp.float32),
            pltpu.VMEM((2 * L,), jnp.float32),
            pltpu.SemaphoreType.DMA,
            pltpu.SemaphoreType.DMA,
            pltpu.SemaphoreType.DMA,
            pltpu.SemaphoreType.DMA,
        ],
    )(_sc_partial_sums)


def _sc_partial_sums(
    p_hbm, t_hbm, v_hbm, out_hbm, p0, p1, t0, t1, v_v, part_v, s0, s1, s2, s3
):
    wid = lax.axis_index("s") * NC + lax.axis_index("c")
    col0 = wid * COLS_W
    pltpu.sync_copy(v_hbm, v_v)
    v = v_v[...]
    bufs = ((p0, t0, s0, s1), (p1, t1, s2, s3))

    def start(c):
        pb, tb, sp, st = bufs[c % 2]
        cols = pl.ds(col0 + c * CCHUNK, CCHUNK)
        cp = pltpu.make_async_copy(p_hbm.at[:, cols], pb, sp)
        ct = pltpu.make_async_copy(t_hbm.at[:, cols], tb, st)
        cp.start()
        ct.start()
        return cp, ct

    NACC = 8
    nums = [jnp.zeros((L,), jnp.float32) for _ in range(NACC)]
    cnts = [jnp.zeros((L,), jnp.int32) for _ in range(NACC)]
    pending = start(0)
    for c in range(NCHUNK):
        if c + 1 < NCHUNK:
            nxt = start(c + 1)
        pending[0].wait()
        pending[1].wait()
        pb, tb, _, _ = bufs[c % 2]

        def quad(i, carry, pb=pb, tb=tb):
            accs = list(carry)
            for rr in range(RQUAD):
                r = i * RQUAD + rr
                for j in range(VPR):
                    t = tb[r, pl.ds(j * L, L)]
                    p = pb[r, pl.ds(j * L, L)]
                    m = t > v
                    safe = jnp.where(m, t, jnp.inf)
                    k = j % NACC
                    accs[k] = accs[k] + jnp.abs((t - p) / safe)
                    accs[NACC + k] = accs[NACC + k] + jnp.where(m, 1, 0)
            return tuple(accs)

        out_accs = lax.fori_loop(
            0, ROWS_T // RQUAD, quad, tuple(nums) + tuple(cnts)
        )
        nums = list(out_accs[:NACC])
        cnts = list(out_accs[NACC:])
        if c + 1 < NCHUNK:
            pending = nxt
    num = ((nums[0] + nums[1]) + (nums[2] + nums[3])) + ((nums[4] + nums[5]) + (nums[6] + nums[7]))
    cnt = ((cnts[0] + cnts[1]) + (cnts[2] + cnts[3])) + ((cnts[4] + cnts[5]) + (cnts[6] + cnts[7]))
    part_v[pl.ds(0, L)] = num
    part_v[pl.ds(L, L)] = cnt.astype(jnp.float32)
    pltpu.sync_copy(part_v, out_hbm.at[wid])


def _combine_body(parts_ref, o_ref):
    x = parts_ref[...]
    num = jnp.sum(x[:, :L])
    cnt = jnp.sum(x[:, L:])
    o_ref[...] = jnp.broadcast_to(num / cnt, (1, 1))


def kernel(preds, targets, value):
    v_vec = jnp.full((L,), jnp.asarray(value, jnp.float32))
    parts = _build_sc_partial_sums()(preds.T, targets.T, v_vec)
    out = pl.pallas_call(
        _combine_body,
        out_shape=jax.ShapeDtypeStruct((1, 1), jnp.float32),
    )(parts)
    return out[0, 0]


# speedup vs baseline: 1.9842x; 1.0067x over previous
import functools

import jax
import jax.numpy as jnp
from jax import lax
from jax.experimental import pallas as pl
from jax.experimental.pallas import tpu as pltpu
from jax.experimental.pallas import tpu_sc as plsc

NC, NS = 2, 16
NW = NC * NS
L = 16
ROWS_T, COLS_T = 200, 16384
COLS_W = COLS_T // NW
CCHUNK = 128
NCHUNK = COLS_W // CCHUNK
RQUAD = 2
VPR = CCHUNK // L


@functools.cache
def _build_sc_partial_sums():
    mesh = plsc.VectorSubcoreMesh(
        core_axis_name="c", subcore_axis_name="s", num_cores=NC, num_subcores=NS
    )
    return functools.partial(
        pl.kernel,
        out_type=jax.ShapeDtypeStruct((NW, 2 * L), jnp.float32),
        mesh=mesh,
        compiler_params=pltpu.CompilerParams(skip_device_barrier=True),
        scratch_types=[
            pltpu.VMEM((ROWS_T, CCHUNK), jnp.float32),
            pltpu.VMEM((ROWS_T, CCHUNK), jnp.float32),
            pltpu.VMEM((ROWS_T, CCHUNK), jnp.float32),
            pltpu.VMEM((ROWS_T, CCHUNK), jnp.float32),
            pltpu.VMEM((L,), jnp.float32),
            pltpu.VMEM((2 * L,), jnp.float32),
            pltpu.SemaphoreType.DMA,
            pltpu.SemaphoreType.DMA,
            pltpu.SemaphoreType.DMA,
            pltpu.SemaphoreType.DMA,
        ],
    )(_sc_partial_sums)


def _sc_partial_sums(
    p_hbm, t_hbm, v_hbm, out_hbm, p0, p1, t0, t1, v_v, part_v, s0, s1, s2, s3
):
    wid = lax.axis_index("s") * NC + lax.axis_index("c")
    col0 = wid * COLS_W
    pltpu.sync_copy(v_hbm, v_v)
    v = v_v[...]
    bufs = ((p0, t0, s0, s1), (p1, t1, s2, s3))

    def start(c):
        pb, tb, sp, st = bufs[c % 2]
        cols = pl.ds(col0 + c * CCHUNK, CCHUNK)
        cp = pltpu.make_async_copy(p_hbm.at[:, cols], pb, sp)
        ct = pltpu.make_async_copy(t_hbm.at[:, cols], tb, st)
        cp.start()
        ct.start()
        return cp, ct

    NACC = 8
    nums = [jnp.zeros((L,), jnp.float32) for _ in range(NACC)]
    cnts = [jnp.zeros((L,), jnp.int32) for _ in range(NACC)]
    pending = start(0)
    for c in range(NCHUNK):
        if c + 1 < NCHUNK:
            nxt = start(c + 1)
        pending[0].wait()
        pending[1].wait()
        pb, tb, _, _ = bufs[c % 2]

        def quad(i, carry, pb=pb, tb=tb):
            accs = list(carry)
            for rr in range(RQUAD):
                r = i * RQUAD + rr
                for j in range(VPR):
                    t = tb[r, pl.ds(j * L, L)]
                    p = pb[r, pl.ds(j * L, L)]
                    m = t > v
                    safe = jnp.where(m, t, jnp.inf)
                    k = j % NACC
                    accs[k] = accs[k] + jnp.abs((t - p) / safe)
                    accs[NACC + k] = accs[NACC + k] + jnp.where(m, 1, 0)
            return tuple(accs)

        out_accs = lax.fori_loop(
            0, ROWS_T // RQUAD, quad, tuple(nums) + tuple(cnts)
        )
        nums = list(out_accs[:NACC])
        cnts = list(out_accs[NACC:])
        if c + 1 < NCHUNK:
            pending = nxt
    num = ((nums[0] + nums[1]) + (nums[2] + nums[3])) + ((nums[4] + nums[5]) + (nums[6] + nums[7]))
    cnt = ((cnts[0] + cnts[1]) + (cnts[2] + cnts[3])) + ((cnts[4] + cnts[5]) + (cnts[6] + cnts[7]))
    part_v[pl.ds(0, L)] = num
    part_v[pl.ds(L, L)] = cnt.astype(jnp.float32)
    pltpu.sync_copy(part_v, out_hbm.at[wid])


def _combine_body(parts_ref, o_ref):
    x = parts_ref[...]
    num = jnp.sum(x[:, :L])
    cnt = jnp.sum(x[:, L:])
    o_ref[...] = jnp.broadcast_to(num / cnt, (1, 1))


def kernel(preds, targets, value):
    v_vec = jnp.full((L,), jnp.asarray(value, jnp.float32))
    parts = _build_sc_partial_sums()(preds.T, targets.T, v_vec)
    out = pl.pallas_call(
        _combine_body,
        out_shape=jax.ShapeDtypeStruct((1, 1), jnp.float32),
    )(parts)
    return out[0, 0]


# speedup vs baseline: 2.1943x; 1.1059x over previous
import functools

import jax
import jax.numpy as jnp
from jax import lax
from jax.experimental import pallas as pl
from jax.experimental.pallas import tpu as pltpu
from jax.experimental.pallas import tpu_sc as plsc

NC, NS = 2, 16
NW = NC * NS
L = 16
ROWS_T, COLS_T = 200, 16384
C_SC = 8192
COLS_W = C_SC // NW
CCHUNK = 128
NCHUNK = COLS_W // CCHUNK
RQUAD = 2
VPR = CCHUNK // L
NACC = 4
TBLK = 2048


@functools.cache
def _build_sc_partial_sums():
    mesh = plsc.VectorSubcoreMesh(
        core_axis_name="c", subcore_axis_name="s", num_cores=NC, num_subcores=NS
    )
    return functools.partial(
        pl.kernel,
        out_type=jax.ShapeDtypeStruct((NW, 2 * L), jnp.float32),
        mesh=mesh,
        scratch_types=[
            pltpu.VMEM((ROWS_T, CCHUNK), jnp.float32),
            pltpu.VMEM((ROWS_T, CCHUNK), jnp.float32),
            pltpu.VMEM((ROWS_T, CCHUNK), jnp.float32),
            pltpu.VMEM((ROWS_T, CCHUNK), jnp.float32),
            pltpu.VMEM((L,), jnp.float32),
            pltpu.VMEM((2 * L,), jnp.float32),
            pltpu.SemaphoreType.DMA,
            pltpu.SemaphoreType.DMA,
            pltpu.SemaphoreType.DMA,
            pltpu.SemaphoreType.DMA,
        ],
    )(_sc_partial_sums)


def _sc_partial_sums(
    p_hbm, t_hbm, v_hbm, out_hbm, p0, p1, t0, t1, v_v, part_v, s0, s1, s2, s3
):
    wid = lax.axis_index("s") * NC + lax.axis_index("c")
    col0 = wid * COLS_W
    pltpu.sync_copy(v_hbm, v_v)
    v = v_v[...]
    bufs = ((p0, t0, s0, s1), (p1, t1, s2, s3))

    def start(c):
        pb, tb, sp, st = bufs[c % 2]
        cols = pl.ds(col0 + c * CCHUNK, CCHUNK)
        cp = pltpu.make_async_copy(p_hbm.at[:, cols], pb, sp)
        ct = pltpu.make_async_copy(t_hbm.at[:, cols], tb, st)
        cp.start()
        ct.start()
        return cp, ct

    nums = [jnp.zeros((L,), jnp.float32) for _ in range(NACC)]
    cnts = [jnp.zeros((L,), jnp.int32) for _ in range(NACC)]
    pending = start(0)
    for c in range(NCHUNK):
        if c + 1 < NCHUNK:
            nxt = start(c + 1)
        pending[0].wait()
        pending[1].wait()
        pb, tb, _, _ = bufs[c % 2]

        def quad(i, carry, pb=pb, tb=tb):
            accs = list(carry)
            for rr in range(RQUAD):
                r = i * RQUAD + rr
                for j in range(VPR):
                    t = tb[r, pl.ds(j * L, L)]
                    p = pb[r, pl.ds(j * L, L)]
                    m = t > v
                    safe = jnp.where(m, t, jnp.inf)
                    k = j % NACC
                    accs[k] = accs[k] + jnp.abs((t - p) / safe)
                    accs[NACC + k] = accs[NACC + k] + jnp.where(m, 1, 0)
            return tuple(accs)

        out_accs = lax.fori_loop(
            0, ROWS_T // RQUAD, quad, tuple(nums) + tuple(cnts)
        )
        nums = list(out_accs[:NACC])
        cnts = list(out_accs[NACC:])
        if c + 1 < NCHUNK:
            pending = nxt
    num = (nums[0] + nums[1]) + (nums[2] + nums[3])
    cnt = (cnts[0] + cnts[1]) + (cnts[2] + cnts[3])
    part_v[pl.ds(0, L)] = num
    part_v[pl.ds(L, L)] = cnt.astype(jnp.float32)
    pltpu.sync_copy(part_v, out_hbm.at[wid])


def _tc_partial_body(v_ref, p_ref, t_ref, o_ref):
    i = pl.program_id(0)
    v = v_ref[0]
    t = t_ref[...]
    p = p_ref[...]
    m = t > v
    safe = jnp.where(m, t, jnp.inf)
    num = jnp.sum(jnp.abs((t - p) / safe))
    cnt = jnp.sum(m.astype(jnp.float32))

    @pl.when(i == 0)
    def _():
        o_ref[0] = 0.0
        o_ref[1] = 0.0

    o_ref[0] += num
    o_ref[1] += cnt


def _combine_body(sc_ref, tc_ref, o_ref):
    x = sc_ref[...]
    num = jnp.sum(x[:, :L]) + tc_ref[0]
    cnt = jnp.sum(x[:, L:]) + tc_ref[1]
    o_ref[0, 0] = num / cnt


def kernel(preds, targets, value):
    v_f32 = jnp.asarray(value, jnp.float32)
    v_vec = jnp.full((L,), v_f32)
    pT = preds.T
    tT = targets.T
    sc_parts = _build_sc_partial_sums()(pT, tT, v_vec)
    tc_parts = pl.pallas_call(
        _tc_partial_body,
        grid=((COLS_T - C_SC) // TBLK,),
        in_specs=[
            pl.BlockSpec(memory_space=pltpu.SMEM),
            pl.BlockSpec((ROWS_T, TBLK), lambda i: (0, C_SC // TBLK + i)),
            pl.BlockSpec((ROWS_T, TBLK), lambda i: (0, C_SC // TBLK + i)),
        ],
        out_specs=pl.BlockSpec(memory_space=pltpu.SMEM),
        out_shape=jax.ShapeDtypeStruct((2,), jnp.float32),
    )(jnp.reshape(v_f32, (1,)), pT, tT)
    out = pl.pallas_call(
        _combine_body,
        in_specs=[
            pl.BlockSpec(memory_space=pltpu.VMEM),
            pl.BlockSpec(memory_space=pltpu.SMEM),
        ],
        out_specs=pl.BlockSpec(memory_space=pltpu.SMEM),
        out_shape=jax.ShapeDtypeStruct((1, 1), jnp.float32),
    )(sc_parts, tc_parts)
    return out[0, 0]


# speedup vs baseline: 2.5208x; 1.1488x over previous
import functools

import jax
import jax.numpy as jnp
from jax import lax
from jax.experimental import pallas as pl
from jax.experimental.pallas import tpu as pltpu
from jax.experimental.pallas import tpu_sc as plsc

NC, NS = 2, 16
NW = NC * NS
L = 16
ROWS_T, COLS_T = 200, 16384
C_SC = 4096
COLS_W = C_SC // NW
CCHUNK = 128
NCHUNK = COLS_W // CCHUNK
RQUAD = 2
VPR = CCHUNK // L
NACC = 4
TBLK = 2048


@functools.cache
def _build_sc_partial_sums():
    mesh = plsc.VectorSubcoreMesh(
        core_axis_name="c", subcore_axis_name="s", num_cores=NC, num_subcores=NS
    )
    return functools.partial(
        pl.kernel,
        out_type=jax.ShapeDtypeStruct((NW, 2 * L), jnp.float32),
        mesh=mesh,
        scratch_types=[
            pltpu.VMEM((ROWS_T, CCHUNK), jnp.float32),
            pltpu.VMEM((ROWS_T, CCHUNK), jnp.float32),
            pltpu.VMEM((ROWS_T, CCHUNK), jnp.float32),
            pltpu.VMEM((ROWS_T, CCHUNK), jnp.float32),
            pltpu.VMEM((L,), jnp.float32),
            pltpu.VMEM((2 * L,), jnp.float32),
            pltpu.SemaphoreType.DMA,
            pltpu.SemaphoreType.DMA,
            pltpu.SemaphoreType.DMA,
            pltpu.SemaphoreType.DMA,
        ],
    )(_sc_partial_sums)


def _sc_partial_sums(
    p_hbm, t_hbm, v_hbm, out_hbm, p0, p1, t0, t1, v_v, part_v, s0, s1, s2, s3
):
    wid = lax.axis_index("s") * NC + lax.axis_index("c")
    col0 = wid * COLS_W
    pltpu.sync_copy(v_hbm, v_v)
    v = v_v[...]
    bufs = ((p0, t0, s0, s1), (p1, t1, s2, s3))

    def start(c):
        pb, tb, sp, st = bufs[c % 2]
        cols = pl.ds(col0 + c * CCHUNK, CCHUNK)
        cp = pltpu.make_async_copy(p_hbm.at[:, cols], pb, sp)
        ct = pltpu.make_async_copy(t_hbm.at[:, cols], tb, st)
        cp.start()
        ct.start()
        return cp, ct

    nums = [jnp.zeros((L,), jnp.float32) for _ in range(NACC)]
    cnts = [jnp.zeros((L,), jnp.int32) for _ in range(NACC)]
    pending = start(0)
    for c in range(NCHUNK):
        if c + 1 < NCHUNK:
            nxt = start(c + 1)
        pending[0].wait()
        pending[1].wait()
        pb, tb, _, _ = bufs[c % 2]

        def quad(i, carry, pb=pb, tb=tb):
            accs = list(carry)
            for rr in range(RQUAD):
                r = i * RQUAD + rr
                for j in range(VPR):
                    t = tb[r, pl.ds(j * L, L)]
                    p = pb[r, pl.ds(j * L, L)]
                    m = t > v
                    safe = jnp.where(m, t, jnp.inf)
                    k = j % NACC
                    accs[k] = accs[k] + jnp.abs((t - p) / safe)
                    accs[NACC + k] = accs[NACC + k] + jnp.where(m, 1, 0)
            return tuple(accs)

        out_accs = lax.fori_loop(
            0, ROWS_T // RQUAD, quad, tuple(nums) + tuple(cnts)
        )
        nums = list(out_accs[:NACC])
        cnts = list(out_accs[NACC:])
        if c + 1 < NCHUNK:
            pending = nxt
    num = (nums[0] + nums[1]) + (nums[2] + nums[3])
    cnt = (cnts[0] + cnts[1]) + (cnts[2] + cnts[3])
    part_v[pl.ds(0, L)] = num
    part_v[pl.ds(L, L)] = cnt.astype(jnp.float32)
    pltpu.sync_copy(part_v, out_hbm.at[wid])


def _tc_partial_body(v_ref, p_ref, t_ref, o_ref):
    i = pl.program_id(0)
    v = v_ref[0]
    t = t_ref[...]
    p = p_ref[...]
    m = t > v
    safe = jnp.where(m, t, jnp.inf)
    num = jnp.sum(jnp.abs((t - p) / safe))
    cnt = jnp.sum(m.astype(jnp.float32))

    @pl.when(i == 0)
    def _():
        o_ref[0] = 0.0
        o_ref[1] = 0.0

    o_ref[0] += num
    o_ref[1] += cnt


def _combine_body(sc_ref, tc_ref, o_ref):
    x = sc_ref[...]
    num = jnp.sum(x[:, :L]) + tc_ref[0]
    cnt = jnp.sum(x[:, L:]) + tc_ref[1]
    o_ref[0, 0] = num / cnt


def kernel(preds, targets, value):
    v_f32 = jnp.asarray(value, jnp.float32)
    v_vec = jnp.full((L,), v_f32)
    pT = preds.T
    tT = targets.T
    sc_parts = _build_sc_partial_sums()(pT, tT, v_vec)
    tc_parts = pl.pallas_call(
        _tc_partial_body,
        grid=((COLS_T - C_SC) // TBLK,),
        in_specs=[
            pl.BlockSpec(memory_space=pltpu.SMEM),
            pl.BlockSpec((ROWS_T, TBLK), lambda i: (0, C_SC // TBLK + i)),
            pl.BlockSpec((ROWS_T, TBLK), lambda i: (0, C_SC // TBLK + i)),
        ],
        out_specs=pl.BlockSpec(memory_space=pltpu.SMEM),
        out_shape=jax.ShapeDtypeStruct((2,), jnp.float32),
    )(jnp.reshape(v_f32, (1,)), pT, tT)
    out = pl.pallas_call(
        _combine_body,
        in_specs=[
            pl.BlockSpec(memory_space=pltpu.VMEM),
            pl.BlockSpec(memory_space=pltpu.SMEM),
        ],
        out_specs=pl.BlockSpec(memory_space=pltpu.SMEM),
        out_shape=jax.ShapeDtypeStruct((1, 1), jnp.float32),
    )(sc_parts, tc_parts)
    return out[0, 0]


# speedup vs baseline: 2.5387x; 1.0071x over previous
import functools

import jax
import jax.numpy as jnp
from jax import lax
from jax.experimental import pallas as pl
from jax.experimental.pallas import tpu as pltpu
from jax.experimental.pallas import tpu_sc as plsc

NC, NS = 2, 16
NW = NC * NS
L = 16
ROWS_T, COLS_T = 200, 16384
C_SC = 4096
COLS_W = C_SC // NW
CCHUNK = 128
NCHUNK = COLS_W // CCHUNK
RSPLIT = (96, 104)
RQUAD = 2
VPR = CCHUNK // L
NACC = 4
TBLK = 2048


@functools.cache
def _build_sc_partial_sums():
    mesh = plsc.VectorSubcoreMesh(
        core_axis_name="c", subcore_axis_name="s", num_cores=NC, num_subcores=NS
    )
    return functools.partial(
        pl.kernel,
        out_type=jax.ShapeDtypeStruct((NW, 2 * L), jnp.float32),
        mesh=mesh,
        scratch_types=[
            pltpu.VMEM((ROWS_T, CCHUNK), jnp.float32),
            pltpu.VMEM((ROWS_T, CCHUNK), jnp.float32),
            pltpu.VMEM((ROWS_T, CCHUNK), jnp.float32),
            pltpu.VMEM((ROWS_T, CCHUNK), jnp.float32),
            pltpu.VMEM((L,), jnp.float32),
            pltpu.VMEM((2 * L,), jnp.float32),
            pltpu.SemaphoreType.DMA,
            pltpu.SemaphoreType.DMA,
            pltpu.SemaphoreType.DMA,
            pltpu.SemaphoreType.DMA,
        ],
    )(_sc_partial_sums)


def _sc_partial_sums(
    p_hbm, t_hbm, v_hbm, out_hbm, p0, p1, t0, t1, v_v, part_v, s0, s1, s2, s3
):
    wid = lax.axis_index("s") * NC + lax.axis_index("c")
    col0 = wid * COLS_W
    cols = pl.ds(col0, CCHUNK)
    half = []
    r0 = 0
    sems = ((s0, s1), (s2, s3))
    for h, nrows in enumerate(RSPLIT):
        rows = pl.ds(r0, nrows)
        sp, st = sems[h]
        cp = pltpu.make_async_copy(p_hbm.at[rows, cols], p0.at[rows], sp)
        ct = pltpu.make_async_copy(t_hbm.at[rows, cols], t0.at[rows], st)
        cp.start()
        ct.start()
        half.append((r0, nrows, cp, ct))
        r0 += nrows
    pltpu.sync_copy(v_hbm, v_v)
    v = v_v[...]

    nums = [jnp.zeros((L,), jnp.float32) for _ in range(NACC)]
    cnts = [jnp.zeros((L,), jnp.int32) for _ in range(NACC)]
    for r0, nrows, cp, ct in half:
        cp.wait()
        ct.wait()

        def quad(i, carry, r0=r0):
            accs = list(carry)
            for rr in range(RQUAD):
                r = r0 + i * RQUAD + rr
                for j in range(VPR):
                    t = t0[r, pl.ds(j * L, L)]
                    p = p0[r, pl.ds(j * L, L)]
                    m = t > v
                    safe = jnp.where(m, t, jnp.inf)
                    k = j % NACC
                    accs[k] = accs[k] + jnp.abs((t - p) / safe)
                    accs[NACC + k] = accs[NACC + k] + jnp.where(m, 1, 0)
            return tuple(accs)

        out_accs = lax.fori_loop(
            0, nrows // RQUAD, quad, tuple(nums) + tuple(cnts)
        )
        nums = list(out_accs[:NACC])
        cnts = list(out_accs[NACC:])
    num = (nums[0] + nums[1]) + (nums[2] + nums[3])
    cnt = (cnts[0] + cnts[1]) + (cnts[2] + cnts[3])
    part_v[pl.ds(0, L)] = num
    part_v[pl.ds(L, L)] = cnt.astype(jnp.float32)
    pltpu.sync_copy(part_v, out_hbm.at[wid])


def _tc_partial_body(v_ref, p_ref, t_ref, o_ref):
    i = pl.program_id(0)
    v = v_ref[0]
    t = t_ref[...]
    p = p_ref[...]
    m = t > v
    safe = jnp.where(m, t, jnp.inf)
    num = jnp.sum(jnp.abs((t - p) / safe))
    cnt = jnp.sum(m.astype(jnp.float32))

    @pl.when(i == 0)
    def _():
        o_ref[0] = 0.0
        o_ref[1] = 0.0

    o_ref[0] += num
    o_ref[1] += cnt


def _combine_body(sc_ref, tc_ref, o_ref):
    x = sc_ref[...]
    num = jnp.sum(x[:, :L]) + tc_ref[0]
    cnt = jnp.sum(x[:, L:]) + tc_ref[1]
    o_ref[0, 0] = num / cnt


def kernel(preds, targets, value):
    v_f32 = jnp.asarray(value, jnp.float32)
    v_vec = jnp.full((L,), v_f32)
    pT = preds.T
    tT = targets.T
    sc_parts = _build_sc_partial_sums()(pT, tT, v_vec)
    tc_parts = pl.pallas_call(
        _tc_partial_body,
        grid=((COLS_T - C_SC) // TBLK,),
        in_specs=[
            pl.BlockSpec(memory_space=pltpu.SMEM),
            pl.BlockSpec((ROWS_T, TBLK), lambda i: (0, C_SC // TBLK + i)),
            pl.BlockSpec((ROWS_T, TBLK), lambda i: (0, C_SC // TBLK + i)),
        ],
        out_specs=pl.BlockSpec(memory_space=pltpu.SMEM),
        out_shape=jax.ShapeDtypeStruct((2,), jnp.float32),
    )(jnp.reshape(v_f32, (1,)), pT, tT)
    out = pl.pallas_call(
        _combine_body,
        in_specs=[
            pl.BlockSpec(memory_space=pltpu.VMEM),
            pl.BlockSpec(memory_space=pltpu.SMEM),
        ],
        out_specs=pl.BlockSpec(memory_space=pltpu.SMEM),
        out_shape=jax.ShapeDtypeStruct((1, 1), jnp.float32),
    )(sc_parts, tc_parts)
    return out[0, 0]


# speedup vs baseline: 2.5853x; 1.0183x over previous
import functools

import jax
import jax.numpy as jnp
from jax import lax
from jax.experimental import pallas as pl
from jax.experimental.pallas import tpu as pltpu
from jax.experimental.pallas import tpu_sc as plsc

NC, NS = 2, 16
NW = NC * NS
L = 16
ROWS_T, COLS_T = 200, 16384
C_SC = 4096
COLS_W = C_SC // NW
CCHUNK = 128
NCHUNK = COLS_W // CCHUNK
RSPLIT = (96, 104)
RQUAD = 2
VPR = CCHUNK // L
NACC = 4
TBLK = 4096


@functools.cache
def _build_sc_partial_sums():
    mesh = plsc.VectorSubcoreMesh(
        core_axis_name="c", subcore_axis_name="s", num_cores=NC, num_subcores=NS
    )
    return functools.partial(
        pl.kernel,
        out_type=jax.ShapeDtypeStruct((NW, 2 * L), jnp.float32),
        mesh=mesh,
        scratch_types=[
            pltpu.VMEM((ROWS_T, CCHUNK), jnp.float32),
            pltpu.VMEM((ROWS_T, CCHUNK), jnp.float32),
            pltpu.VMEM((L,), jnp.float32),
            pltpu.VMEM((2 * L,), jnp.float32),
            pltpu.SemaphoreType.DMA,
            pltpu.SemaphoreType.DMA,
            pltpu.SemaphoreType.DMA,
            pltpu.SemaphoreType.DMA,
        ],
    )(_sc_partial_sums)


def _sc_partial_sums(
    p_hbm, t_hbm, v_hbm, out_hbm, p0, t0, v_v, part_v, s0, s1, s2, s3
):
    wid = lax.axis_index("s") * NC + lax.axis_index("c")
    col0 = wid * COLS_W
    cols = pl.ds(col0, CCHUNK)
    half = []
    r0 = 0
    sems = ((s0, s1), (s2, s3))
    for h, nrows in enumerate(RSPLIT):
        rows = pl.ds(r0, nrows)
        sp, st = sems[h]
        cp = pltpu.make_async_copy(p_hbm.at[rows, cols], p0.at[rows], sp)
        ct = pltpu.make_async_copy(t_hbm.at[rows, cols], t0.at[rows], st)
        cp.start()
        ct.start()
        half.append((r0, nrows, cp, ct))
        r0 += nrows
    pltpu.sync_copy(v_hbm, v_v)
    v = v_v[...]

    nums = [jnp.zeros((L,), jnp.float32) for _ in range(NACC)]
    cnts = [jnp.zeros((L,), jnp.int32) for _ in range(NACC)]
    for r0, nrows, cp, ct in half:
        cp.wait()
        ct.wait()

        def quad(i, carry, r0=r0):
            accs = list(carry)
            for rr in range(RQUAD):
                r = r0 + i * RQUAD + rr
                for j in range(VPR):
                    t = t0[r, pl.ds(j * L, L)]
                    p = p0[r, pl.ds(j * L, L)]
                    m = t > v
                    safe = jnp.where(m, t, jnp.inf)
                    k = j % NACC
                    accs[k] = accs[k] + jnp.abs((t - p) / safe)
                    accs[NACC + k] = accs[NACC + k] + jnp.where(m, 1, 0)
            return tuple(accs)

        out_accs = lax.fori_loop(
            0, nrows // RQUAD, quad, tuple(nums) + tuple(cnts)
        )
        nums = list(out_accs[:NACC])
        cnts = list(out_accs[NACC:])
    num = (nums[0] + nums[1]) + (nums[2] + nums[3])
    cnt = (cnts[0] + cnts[1]) + (cnts[2] + cnts[3])
    part_v[pl.ds(0, L)] = num
    part_v[pl.ds(L, L)] = cnt.astype(jnp.float32)
    pltpu.sync_copy(part_v, out_hbm.at[wid])


def _tc_partial_body(v_ref, p_ref, t_ref, o_ref):
    i = pl.program_id(0)
    v = v_ref[0]
    t = t_ref[...]
    p = p_ref[...]
    m = t > v
    safe = jnp.where(m, t, jnp.inf)
    num = jnp.sum(jnp.abs((t - p) / safe))
    cnt = jnp.sum(m.astype(jnp.float32))

    @pl.when(i == 0)
    def _():
        o_ref[0] = 0.0
        o_ref[1] = 0.0

    o_ref[0] += num
    o_ref[1] += cnt


def _combine_body(sc_ref, tc_ref, o_ref):
    x = sc_ref[...]
    num = jnp.sum(x[:, :L]) + tc_ref[0]
    cnt = jnp.sum(x[:, L:]) + tc_ref[1]
    o_ref[0, 0] = num / cnt


def kernel(preds, targets, value):
    v_f32 = jnp.asarray(value, jnp.float32)
    v_vec = jnp.full((L,), v_f32)
    pT = preds.T
    tT = targets.T
    sc_parts = _build_sc_partial_sums()(pT, tT, v_vec)
    tc_parts = pl.pallas_call(
        _tc_partial_body,
        grid=((COLS_T - C_SC) // TBLK,),
        in_specs=[
            pl.BlockSpec(memory_space=pltpu.SMEM),
            pl.BlockSpec((ROWS_T, TBLK), lambda i: (0, C_SC // TBLK + i)),
            pl.BlockSpec((ROWS_T, TBLK), lambda i: (0, C_SC // TBLK + i)),
        ],
        out_specs=pl.BlockSpec(memory_space=pltpu.SMEM),
        out_shape=jax.ShapeDtypeStruct((2,), jnp.float32),
    )(jnp.reshape(v_f32, (1,)), pT, tT)
    out = pl.pallas_call(
        _combine_body,
        in_specs=[
            pl.BlockSpec(memory_space=pltpu.VMEM),
            pl.BlockSpec(memory_space=pltpu.SMEM),
        ],
        out_specs=pl.BlockSpec(memory_space=pltpu.SMEM),
        out_shape=jax.ShapeDtypeStruct((1, 1), jnp.float32),
    )(sc_parts, tc_parts)
    return out[0, 0]


# speedup vs baseline: 2.6108x; 1.0099x over previous
import functools

import jax
import jax.numpy as jnp
from jax import lax
from jax.experimental import pallas as pl
from jax.experimental.pallas import tpu as pltpu
from jax.experimental.pallas import tpu_sc as plsc

NC, NS = 2, 16
NW = NC * NS
L = 16
ROWS_T, COLS_T = 200, 16384
C_SC = 4096
COLS_W = C_SC // NW
CCHUNK = 128
NCHUNK = COLS_W // CCHUNK
RSPLIT = (96, 104)
RQUAD = 2
VPR = CCHUNK // L
NACC = 4
TBLK = 4096


@functools.cache
def _build_sc_partial_sums():
    mesh = plsc.VectorSubcoreMesh(
        core_axis_name="c", subcore_axis_name="s", num_cores=NC, num_subcores=NS
    )
    return functools.partial(
        pl.kernel,
        out_type=jax.ShapeDtypeStruct((NW, 2 * L), jnp.float32),
        mesh=mesh,
        scratch_types=[
            pltpu.VMEM((ROWS_T, CCHUNK), jnp.float32),
            pltpu.VMEM((ROWS_T, CCHUNK), jnp.float32),
            pltpu.VMEM((L,), jnp.float32),
            pltpu.VMEM((2 * L,), jnp.float32),
            pltpu.SemaphoreType.DMA,
            pltpu.SemaphoreType.DMA,
            pltpu.SemaphoreType.DMA,
            pltpu.SemaphoreType.DMA,
        ],
    )(_sc_partial_sums)


def _sc_partial_sums(
    p_hbm, t_hbm, v_hbm, out_hbm, p0, t0, v_v, part_v, s0, s1, s2, s3
):
    wid = lax.axis_index("s") * NC + lax.axis_index("c")
    col0 = wid * COLS_W
    cols = pl.ds(col0, CCHUNK)
    half = []
    r0 = 0
    sems = ((s0, s1), (s2, s3))
    for h, nrows in enumerate(RSPLIT):
        rows = pl.ds(r0, nrows)
        sp, st = sems[h]
        cp = pltpu.make_async_copy(p_hbm.at[rows, cols], p0.at[rows], sp)
        ct = pltpu.make_async_copy(t_hbm.at[rows, cols], t0.at[rows], st)
        cp.start()
        ct.start()
        half.append((r0, nrows, cp, ct))
        r0 += nrows
    pltpu.sync_copy(v_hbm, v_v)
    v = v_v[...]

    nums = [jnp.zeros((L,), jnp.float32) for _ in range(NACC)]
    cnts = [jnp.zeros((L,), jnp.int32) for _ in range(NACC)]
    for r0, nrows, cp, ct in half:
        cp.wait()
        ct.wait()

        def quad(i, carry, r0=r0):
            accs = list(carry)
            for rr in range(RQUAD):
                r = r0 + i * RQUAD + rr
                for j in range(VPR):
                    t = t0[r, pl.ds(j * L, L)]
                    p = p0[r, pl.ds(j * L, L)]
                    m = t > v
                    safe = jnp.where(m, t, jnp.inf)
                    k = j % NACC
                    accs[k] = accs[k] + jnp.abs((t - p) / safe)
                    accs[NACC + k] = accs[NACC + k] + jnp.where(m, 1, 0)
            return tuple(accs)

        out_accs = lax.fori_loop(
            0, nrows // RQUAD, quad, tuple(nums) + tuple(cnts)
        )
        nums = list(out_accs[:NACC])
        cnts = list(out_accs[NACC:])
    num = (nums[0] + nums[1]) + (nums[2] + nums[3])
    cnt = (cnts[0] + cnts[1]) + (cnts[2] + cnts[3])
    part_v[pl.ds(0, L)] = num
    part_v[pl.ds(L, L)] = cnt.astype(jnp.float32)
    pltpu.sync_copy(part_v, out_hbm.at[wid])


def _tc_partial_body(v_ref, p_ref, t_ref, o_ref):
    i = pl.program_id(0)
    v = v_ref[0]
    t = t_ref[...]
    p = p_ref[...]
    m = t > v
    safe = jnp.where(m, t, jnp.inf)
    num = jnp.sum(jnp.abs((t - p) / safe))
    cnt = jnp.sum(m.astype(jnp.float32))

    @pl.when(i == 0)
    def _():
        o_ref[0] = 0.0
        o_ref[1] = 0.0

    o_ref[0] += num
    o_ref[1] += cnt


def _combine_body(sc_ref, tc_ref, o_ref):
    x = sc_ref[...]
    num = jnp.sum(x[:, :L]) + tc_ref[0]
    cnt = jnp.sum(x[:, L:]) + tc_ref[1]
    o_ref[0, 0] = num / cnt


def kernel(preds, targets, value):
    v_f32 = jnp.asarray(value, jnp.float32)
    v_vec = jnp.full((L,), v_f32)
    pT = preds.T
    tT = targets.T
    tc_parts = pl.pallas_call(
        _tc_partial_body,
        grid=((COLS_T - C_SC) // TBLK,),
        in_specs=[
            pl.BlockSpec(memory_space=pltpu.SMEM),
            pl.BlockSpec((ROWS_T, TBLK), lambda i: (0, C_SC // TBLK + i)),
            pl.BlockSpec((ROWS_T, TBLK), lambda i: (0, C_SC // TBLK + i)),
        ],
        out_specs=pl.BlockSpec(memory_space=pltpu.SMEM),
        out_shape=jax.ShapeDtypeStruct((2,), jnp.float32),
    )(jnp.reshape(v_f32, (1,)), pT, tT)
    sc_parts = _build_sc_partial_sums()(pT, tT, v_vec)
    out = pl.pallas_call(
        _combine_body,
        in_specs=[
            pl.BlockSpec(memory_space=pltpu.VMEM),
            pl.BlockSpec(memory_space=pltpu.SMEM),
        ],
        out_specs=pl.BlockSpec(memory_space=pltpu.SMEM),
        out_shape=jax.ShapeDtypeStruct((1, 1), jnp.float32),
    )(sc_parts, tc_parts)
    return out[0, 0]
